# baseline (speedup 1.0000x reference)
import functools

import jax
import jax.numpy as jnp
from jax import lax
from jax.experimental import pallas as pl
from jax.experimental.pallas import tpu as pltpu
from jax.experimental.pallas import tpu_sc as plsc

_N = 10000
_D = 32
_V = 128
_H = 128
_G3 = 3 * _H

_NC = 5
_NCHUNK = _N // _NC

_NW = 32
_CROWS = _NCHUNK * _D
_PER_W = _CROWS // _NW
_CH = 80


_NCH = _PER_W // _CH


def _sc_gather(matrix, idx_flat):
    mesh = plsc.VectorSubcoreMesh(core_axis_name="c", subcore_axis_name="s")

    @functools.partial(
        pl.kernel,
        mesh=mesh,
        out_type=jax.ShapeDtypeStruct((_CROWS, _V), jnp.float32),
        scratch_types=[
            pltpu.VMEM((_CH,), jnp.int32),
            pltpu.VMEM((_CH, _V), jnp.float32),
            pltpu.SemaphoreType.DMA,
        ],
    )
    def gather_k(mat_hbm, idx_hbm, out_hbm, idx_v, rows_v, sem):
        wid = lax.axis_index("s") * 2 + lax.axis_index("c")
        base = wid * _PER_W

        def body(i, carry):
            off = base + i * _CH
            pltpu.sync_copy(idx_hbm.at[pl.ds(off, _CH)], idx_v)
            pltpu.async_copy(mat_hbm.at[idx_v], rows_v, sem).wait()
            pltpu.sync_copy(rows_v, out_hbm.at[pl.ds(off, _CH)])
            return carry

        lax.fori_loop(0, _NCH, body, 0)

    return gather_k(matrix, idx_flat)


_B = 400
_BD = _B * _D


def _tc_body(x_ref, cf_ref, wv_ref, wc4_ref, whh_ref, bhhn_ref,
             wot_ref, bo_ref, out_ref):
    x = x_ref[...]
    gi = jnp.dot(x, wv_ref[...], preferred_element_type=jnp.float32)
    gi = gi + jnp.dot(cf_ref[...], wc4_ref[...],
                      preferred_element_type=jnp.float32)

    gi3 = gi.reshape(_B, _D, _G3)
    i_r = gi3[:, :, :_H]
    i_z = gi3[:, :, _H:2 * _H]
    i_n = gi3[:, :, 2 * _H:]

    bhhn = bhhn_ref[...]
    bh_n = bhhn[:, 2 * _H:]

    r1 = jax.nn.sigmoid(i_r)
    z1 = jax.nn.sigmoid(i_z)
    n1 = jnp.tanh(i_n + r1 * bh_n)
    hid = jnp.sum(n1 * (1.0 - z1), axis=1) * (1.0 / _D)

    gh = jnp.dot(hid, whh_ref[...], preferred_element_type=jnp.float32) + bhhn
    gh3 = lax.broadcast_in_dim(gh, (_B, _D, _G3), (0, 2))
    r2 = jax.nn.sigmoid(i_r + gh3[:, :, :_H])
    z2 = jax.nn.sigmoid(i_z + gh3[:, :, _H:2 * _H])
    n2 = jnp.tanh(i_n + r2 * gh3[:, :, 2 * _H:])
    hid3 = lax.broadcast_in_dim(hid, (_B, _D, _H), (0, 2))
    hid2 = jnp.sum(n2 + z2 * (hid3 - n2), axis=1) * (1.0 / _D)

    out_ref[...] = (jnp.dot(hid2, wot_ref[...], preferred_element_type=jnp.float32)
                    + bo_ref[...])


def _tc_call(gathered, cf, wv, wc4, whh, bhhn, wot, bo, interpret=False):
    grid = _NCHUNK // _B
    full = lambda r, c: pl.BlockSpec((r, c), lambda i: (0, 0))
    return pl.pallas_call(
        _tc_body,
        grid=(grid,),
        in_specs=[
            pl.BlockSpec((_BD, _V), lambda i: (i, 0)),
            pl.BlockSpec((_BD, 4), lambda i: (i, 0)),
            full(_V, _G3),
            full(4, _G3),
            full(_H, _G3),
            full(1, _G3),
            full(_H, _H),
            full(1, _H),
        ],
        out_specs=pl.BlockSpec((_B, _H), lambda i: (i, 0)),
        out_shape=jax.ShapeDtypeStruct((_NCHUNK, _H), jnp.float32),
        compiler_params=pltpu.CompilerParams(
            dimension_semantics=("parallel",)),
        interpret=interpret,
    )(gathered, cf, wv, wc4, whh, bhhn, wot, bo)


def kernel(indices, counts, matrix, W_ih, W_hh, b_ih, b_hh, W_out, b_out):
    idx_flat = indices.reshape(_N * _D)
    c = jnp.log2(counts + 1.0)
    cn = c / jnp.sum(c, axis=1, keepdims=True)
    cf = jnp.stack([c, cn, jnp.ones_like(c), jnp.zeros_like(c)],
                   axis=-1).reshape(_N * _D, 4)
    mask_rz = jnp.concatenate([jnp.ones(2 * _H, jnp.float32),
                               jnp.zeros(_H, jnp.float32)])
    bias_row = b_ih + b_hh * mask_rz
    wc4 = jnp.concatenate([W_ih[:, _V:].T, bias_row.reshape(1, _G3),
                           jnp.zeros((1, _G3), jnp.float32)], axis=0)
    wv = W_ih[:, :_V].T
    whh = W_hh.T
    bhhn = (b_hh * (1.0 - mask_rz)).reshape(1, _G3)
    wot = W_out.T
    bo = b_out.reshape(1, _H)

    outs = []
    for k in range(_NC):
        g = _sc_gather(matrix, lax.slice(idx_flat, (k * _CROWS,),
                                         ((k + 1) * _CROWS,)))
        cfk = lax.slice(cf, (k * _CROWS, 0), ((k + 1) * _CROWS, 4))
        outs.append(_tc_call(g, cfk, wv, wc4, whh, bhhn, wot, bo))
    return jnp.concatenate(outs, axis=0)


# speedup vs baseline: 1.0733x; 1.0733x over previous
import functools

import jax
import jax.numpy as jnp
from jax import lax
from jax.experimental import pallas as pl
from jax.experimental.pallas import tpu as pltpu
from jax.experimental.pallas import tpu_sc as plsc

_N = 10000
_D = 32
_V = 128
_H = 128
_G3 = 3 * _H

_NC = 5
_NCHUNK = _N // _NC

_NW = 32
_CROWS = _NCHUNK * _D
_PER_W = _CROWS // _NW
_CH = 80


_NCH = _PER_W // _CH


def _sc_gather(matrix, idx_flat):
    mesh = plsc.VectorSubcoreMesh(core_axis_name="c", subcore_axis_name="s")

    @functools.partial(
        pl.kernel,
        mesh=mesh,
        out_type=jax.ShapeDtypeStruct((_CROWS, _V), jnp.float32),
        scratch_types=[
            pltpu.VMEM((_CH,), jnp.int32),
            pltpu.VMEM((_CH, _V), jnp.float32),
            pltpu.SemaphoreType.DMA,
        ],
    )
    def gather_k(mat_hbm, idx_hbm, out_hbm, idx_v, rows_v, sem):
        wid = lax.axis_index("s") * 2 + lax.axis_index("c")
        base = wid * _PER_W

        def body(i, carry):
            off = base + i * _CH
            pltpu.sync_copy(idx_hbm.at[pl.ds(off, _CH)], idx_v)
            pltpu.async_copy(mat_hbm.at[idx_v], rows_v, sem).wait()
            pltpu.sync_copy(rows_v, out_hbm.at[pl.ds(off, _CH)])
            return carry

        lax.fori_loop(0, _NCH, body, 0)

    return gather_k(matrix, idx_flat)


_B = 200
_BD = _B * _D


def _tc_body(x_ref, cf_ref, wv_ref, wc4_ref, whh_ref, bhhn_ref,
             wot_ref, bo_ref, out_ref):
    x = x_ref[...]
    gi = jnp.dot(x, wv_ref[...], preferred_element_type=jnp.float32)
    gi = gi + jnp.dot(cf_ref[...], wc4_ref[...],
                      preferred_element_type=jnp.float32)

    gi3 = gi.reshape(_B, _D, _G3)
    i_r = gi3[:, :, :_H]
    i_z = gi3[:, :, _H:2 * _H]
    i_n = gi3[:, :, 2 * _H:]

    bhhn = bhhn_ref[...]
    bh_n = bhhn[:, 2 * _H:]

    r1 = 0.5 + 0.5 * jnp.tanh(0.5 * i_r)
    z1 = 0.5 + 0.5 * jnp.tanh(0.5 * i_z)
    n1 = jnp.tanh(i_n + r1 * bh_n)
    hid = jnp.sum(n1 * (1.0 - z1), axis=1) * (1.0 / _D)

    gh = jnp.dot(hid, whh_ref[...], preferred_element_type=jnp.float32) + bhhn
    gh3 = lax.broadcast_in_dim(gh, (_B, _D, _G3), (0, 2))
    r2 = 0.5 + 0.5 * jnp.tanh(0.5 * (i_r + gh3[:, :, :_H]))
    z2 = 0.5 + 0.5 * jnp.tanh(0.5 * (i_z + gh3[:, :, _H:2 * _H]))
    n2 = jnp.tanh(i_n + r2 * gh3[:, :, 2 * _H:])
    hid3 = lax.broadcast_in_dim(hid, (_B, _D, _H), (0, 2))
    hid2 = jnp.sum(n2 + z2 * (hid3 - n2), axis=1) * (1.0 / _D)

    out_ref[...] = (jnp.dot(hid2, wot_ref[...], preferred_element_type=jnp.float32)
                    + bo_ref[...])


def _tc_call(gathered, cf, wv, wc4, whh, bhhn, wot, bo, interpret=False):
    grid = _NCHUNK // _B
    full = lambda r, c: pl.BlockSpec((r, c), lambda i: (0, 0))
    return pl.pallas_call(
        _tc_body,
        grid=(grid,),
        in_specs=[
            pl.BlockSpec((_BD, _V), lambda i: (i, 0)),
            pl.BlockSpec((_BD, 4), lambda i: (i, 0)),
            full(_V, _G3),
            full(4, _G3),
            full(_H, _G3),
            full(1, _G3),
            full(_H, _H),
            full(1, _H),
        ],
        out_specs=pl.BlockSpec((_B, _H), lambda i: (i, 0)),
        out_shape=jax.ShapeDtypeStruct((_NCHUNK, _H), jnp.float32),
        compiler_params=pltpu.CompilerParams(
            dimension_semantics=("parallel",)),
        interpret=interpret,
    )(gathered, cf, wv, wc4, whh, bhhn, wot, bo)


def kernel(indices, counts, matrix, W_ih, W_hh, b_ih, b_hh, W_out, b_out):
    idx_flat = indices.reshape(_N * _D)
    c = jnp.log2(counts + 1.0)
    cn = c / jnp.sum(c, axis=1, keepdims=True)
    cf = jnp.stack([c, cn, jnp.ones_like(c), jnp.zeros_like(c)],
                   axis=-1).reshape(_N * _D, 4)
    mask_rz = jnp.concatenate([jnp.ones(2 * _H, jnp.float32),
                               jnp.zeros(_H, jnp.float32)])
    bias_row = b_ih + b_hh * mask_rz
    wc4 = jnp.concatenate([W_ih[:, _V:].T, bias_row.reshape(1, _G3),
                           jnp.zeros((1, _G3), jnp.float32)], axis=0)
    wv = W_ih[:, :_V].T
    whh = W_hh.T
    bhhn = (b_hh * (1.0 - mask_rz)).reshape(1, _G3)
    wot = W_out.T
    bo = b_out.reshape(1, _H)

    outs = []
    for k in range(_NC):
        g = _sc_gather(matrix, lax.slice(idx_flat, (k * _CROWS,),
                                         ((k + 1) * _CROWS,)))
        cfk = lax.slice(cf, (k * _CROWS, 0), ((k + 1) * _CROWS, 4))
        outs.append(_tc_call(g, cfk, wv, wc4, whh, bhhn, wot, bo))
    return jnp.concatenate(outs, axis=0)
